# baseline (speedup 1.0000x reference)
import jax
import jax.numpy as jnp
from jax import lax
from jax.experimental import pallas as pl
from jax.experimental.pallas import tpu as pltpu
from jax.experimental.pallas import tpu_sc as plsc

D_EMB = 64
LANES = 128
NUM_WORKERS = 32
SCALE = 8.0
VOCAB = 1000000
N_FULL_BLOCKS = VOCAB // LANES
TAIL_COLS = VOCAB - N_FULL_BLOCKS * LANES
BLOCKS_PER_W = N_FULL_BLOCKS // NUM_WORKERS
N_EXTRA = N_FULL_BLOCKS - BLOCKS_PER_W * NUM_WORKERS
NB = 4

_MESH = plsc.VectorSubcoreMesh(core_axis_name="c", subcore_axis_name="s")


def _worker_id():
    return lax.axis_index("s") * lax.axis_size("c") + lax.axis_index("c")


def _transpose_block(in_ref, out_ref, row_iota, n_pairs):
    @plsc.parallel_loop(0, n_pairs, unroll=8)
    def pair_body(k):
        for p in range(2):
            col = k * 2 + p
            for j in range(D_EMB // 16):
                vals = plsc.load_gather(
                    in_ref, [row_iota[j], jnp.full((16,), col, jnp.int32)]
                )
                out_ref[k, pl.ds(p * D_EMB + j * 16, 16)] = vals * SCALE


def _make_transpose_kernel():
    def body(lut_t, tail2, table, in_bufs, out_bufs, sem_in, sem_out):
        wid = _worker_id()
        row_iota = [lax.iota(jnp.int32, 16) + j * 16 for j in range(D_EMB // 16)]

        def in_desc(vt, b):
            return pltpu.make_async_copy(
                lut_t.at[:, pl.ds(vt * LANES, LANES)],
                in_bufs[b].at[:, pl.ds(0, LANES)], sem_in[b]
            )

        def out_desc(vt, b):
            return pltpu.make_async_copy(
                out_bufs[b],
                table.at[pl.ds(vt * (LANES // 2), LANES // 2)],
                sem_out[b],
            )

        base = wid * BLOCKS_PER_W

        for b in range(NB):
            in_desc(base + b, b).start()

        def outer_body(o, carry):
            for b in range(NB):
                vt = base + o * NB + b
                in_desc(vt, b).wait()

                @pl.when(o > 0)
                def _wait_prev_out():
                    out_desc(vt, b).wait()

                _transpose_block(in_bufs[b], out_bufs[b], row_iota, LANES // 2)
                out_desc(vt, b).start()

                @pl.when(o + 1 < BLOCKS_PER_W // NB)
                def _next_in():
                    in_desc(vt + NB, b).start()
            return carry

        n_outer = BLOCKS_PER_W // NB
        lax.fori_loop(0, n_outer, outer_body, None)

        for b in range(NB):
            out_desc(base + (n_outer - 1) * NB + b, b).wait()

        @pl.when(wid < N_EXTRA)
        def _extra():
            vt = NUM_WORKERS * BLOCKS_PER_W + wid
            pltpu.sync_copy(lut_t.at[:, pl.ds(vt * LANES, LANES)],
                            in_bufs[0].at[:, pl.ds(0, LANES)])
            _transpose_block(in_bufs[0], out_bufs[0], row_iota, LANES // 2)
            pltpu.sync_copy(
                out_bufs[0], table.at[pl.ds(vt * (LANES // 2), LANES // 2)]
            )

        @pl.when(wid == N_EXTRA)
        def _tail():
            pltpu.sync_copy(
                tail2, out_bufs[0].at[pl.ds(0, TAIL_COLS // 2)]
            )
            pltpu.sync_copy(
                out_bufs[0].at[pl.ds(0, TAIL_COLS // 2)],
                table.at[pl.ds(N_FULL_BLOCKS * (LANES // 2), TAIL_COLS // 2)],
            )

    return pl.kernel(
        body,
        out_type=jax.ShapeDtypeStruct((VOCAB // 2, LANES), jnp.float32),
        mesh=_MESH,
        scratch_types=[
            [pltpu.VMEM((D_EMB, LANES + 1), jnp.float32) for _ in range(NB)],
            [pltpu.VMEM((LANES // 2, LANES), jnp.float32) for _ in range(NB)],
            [pltpu.SemaphoreType.DMA for _ in range(NB)],
            [pltpu.SemaphoreType.DMA for _ in range(NB)],
        ],
        compiler_params=pltpu.CompilerParams(needs_layout_passes=False),
    )


def _make_gather_kernel(n_l, n_b):
    NBC = 2
    n_bt = n_b // LANES
    n_groups_total = n_l * n_bt
    per_w = n_groups_total // NUM_WORKERS
    n_outer = per_w // NBC

    def body(x_t, table, out3, idx_bufs, shift_bufs, in_bufs, out_bufs,
             sem_idx, sem_in, sem_out):
        wid = _worker_id()
        row_iota = [lax.iota(jnp.int32, 16) + m * 16 for m in range(8)]

        def coords(g):
            l = g // n_bt
            bt = g % n_bt
            return l, bt

        def idx_desc(g, b):
            l, bt = coords(g)
            return pltpu.make_async_copy(
                x_t.at[l, pl.ds(bt * LANES, LANES)], idx_bufs[b], sem_idx[b]
            )

        def gather_desc(b):
            return pltpu.make_async_copy(
                table.at[shift_bufs[b]],
                in_bufs[b].at[:, pl.ds(0, LANES)], sem_in[b]
            )

        def out_desc(g, b):
            l, bt = coords(g)
            return pltpu.make_async_copy(
                out_bufs[b],
                out3.at[l, :, pl.ds(bt * LANES, LANES)],
                sem_out[b],
            )

        def shift_group(b):
            @plsc.parallel_loop(0, LANES // 16, unroll=8)
            def sh_body(m):
                shift_bufs[b][pl.ds(m * 16, 16)] = jax.lax.shift_right_logical(
                    idx_bufs[b][pl.ds(m * 16, 16)], 1
                )

        base = wid * per_w

        for b in range(NBC):
            idx_desc(base + b, b).start()
        for b in range(NBC):
            idx_desc(base + b, b).wait()
            shift_group(b)
            gather_desc(b).start()

        def extract_group(b):
            in_ref = in_bufs[b]
            out_ref = out_bufs[b]
            idx_ref = idx_bufs[b]

            @plsc.parallel_loop(0, LANES // 16, unroll=2)
            def chunk_body(m):
                par = (idx_ref[pl.ds(m * 16, 16)] & 1) * D_EMB
                rows = row_iota[0] + m * 16
                for d in range(D_EMB):
                    vals = plsc.load_gather(in_ref, [rows, par + d])
                    out_ref[d, pl.ds(m * 16, 16)] = vals

        def outer_body(o, carry):
            for b in range(NBC):
                g = base + o * NBC + b
                gather_desc(b).wait()

                @pl.when(o > 0)
                def _wait_prev_out():
                    out_desc(g, b).wait()

                extract_group(b)
                out_desc(g, b).start()

                @pl.when(o + 1 < n_outer)
                def _next():
                    idx_desc(g + NBC, b).start()
                    idx_desc(g + NBC, b).wait()
                    shift_group(b)
                    gather_desc(b).start()
            return carry

        lax.fori_loop(0, n_outer, outer_body, None)

        for b in range(NBC):
            out_desc(base + (n_outer - 1) * NBC + b, b).wait()

    return pl.kernel(
        body,
        out_type=jax.ShapeDtypeStruct((n_l, D_EMB, n_b), jnp.float32),
        mesh=_MESH,
        scratch_types=[
            [pltpu.VMEM((LANES,), jnp.int32) for _ in range(NBC)],
            [pltpu.VMEM((LANES,), jnp.int32) for _ in range(NBC)],
            [pltpu.VMEM((LANES, LANES + 1), jnp.float32) for _ in range(NBC)],
            [pltpu.VMEM((D_EMB, LANES), jnp.float32) for _ in range(NBC)],
            [pltpu.SemaphoreType.DMA for _ in range(NBC)],
            [pltpu.SemaphoreType.DMA for _ in range(NBC)],
            [pltpu.SemaphoreType.DMA for _ in range(NBC)],
        ],
        compiler_params=pltpu.CompilerParams(needs_layout_passes=False),
    )


@jax.jit
def kernel(x, lut):
    n_b, n_l = x.shape
    x_t = x.T.astype(jnp.int32)
    lut_t = lut.T
    tail2 = (lut[N_FULL_BLOCKS * LANES:] * SCALE).reshape(TAIL_COLS // 2, LANES)
    table = _make_transpose_kernel()(lut_t, tail2)
    out3 = _make_gather_kernel(n_l, n_b)(x_t, table)
    return out3.transpose(2, 0, 1)


# speedup vs baseline: 1.4379x; 1.4379x over previous
import math

import jax
import jax.numpy as jnp
from jax import lax
from jax.experimental import pallas as pl
from jax.experimental.pallas import tpu as pltpu
from jax.experimental.pallas import tpu_sc as plsc

D_EMB = 64
NUM_WORKERS = 32
GROUP = 128
NB = 4
SCALE = math.sqrt(D_EMB)


def _make_sc_kernel(n_idx):
    per_w = n_idx // NUM_WORKERS
    n_groups = per_w // GROUP
    n_outer = n_groups // NB

    mesh = plsc.VectorSubcoreMesh(core_axis_name="c", subcore_axis_name="s")

    def body(x_hbm, lut_hbm, out_hbm, idx_v, in_bufs, out_bufs, sem_in, sem_out):
        nc = lax.axis_size("c")
        wid = lax.axis_index("s") * nc + lax.axis_index("c")
        base = wid * per_w

        pltpu.sync_copy(x_hbm.at[pl.ds(base, per_w)], idx_v)

        def gather_desc(g, b):
            return pltpu.make_async_copy(
                lut_hbm.at[idx_v.at[pl.ds(g * GROUP, GROUP)]],
                in_bufs[b],
                sem_in[b],
            )

        def out_desc(g, b):
            return pltpu.make_async_copy(
                out_bufs[b],
                out_hbm.at[pl.ds(base + g * GROUP, GROUP)],
                sem_out[b],
            )

        for b in range(NB):
            gather_desc(b, b).start()

        def scale_group(b):
            in_ref = in_bufs[b]
            out_ref = out_bufs[b]

            @plsc.parallel_loop(0, GROUP // 4, unroll=4)
            def row_body(r):
                for rr in range(4):
                    row = r * 4 + rr
                    for j in range(D_EMB // 16):
                        out_ref[row, pl.ds(j * 16, 16)] = (
                            in_ref[row, pl.ds(j * 16, 16)] * SCALE
                        )

        def outer_body(o, carry):
            for b in range(NB):
                g = o * NB + b
                gather_desc(g, b).wait()

                @pl.when(o > 0)
                def _wait_prev_out():
                    out_desc(g, b).wait()

                scale_group(b)
                out_desc(g, b).start()

                @pl.when(o + 1 < n_outer)
                def _next_gather():
                    gather_desc(g + NB, b).start()
            return carry

        lax.fori_loop(0, n_outer, outer_body, None)

        for b in range(NB):
            out_desc((n_outer - 1) * NB + b, b).wait()

    grid_kernel = pl.kernel(
        body,
        out_type=jax.ShapeDtypeStruct((n_idx, D_EMB), jnp.float32),
        mesh=mesh,
        scratch_types=[
            pltpu.VMEM((per_w,), jnp.int32),
            [pltpu.VMEM((GROUP, D_EMB), jnp.float32) for _ in range(NB)],
            [pltpu.VMEM((GROUP, D_EMB), jnp.float32) for _ in range(NB)],
            [pltpu.SemaphoreType.DMA for _ in range(NB)],
            [pltpu.SemaphoreType.DMA for _ in range(NB)],
        ],
        compiler_params=pltpu.CompilerParams(use_tc_tiling_on_sc=False),
    )
    return grid_kernel


@jax.jit
def kernel(x, lut):
    b, l = x.shape
    flat_idx = x.reshape(b * l).astype(jnp.int32)
    out = _make_sc_kernel(b * l)(flat_idx, lut)
    return out.reshape(b, l, D_EMB)
